# baseline (speedup 1.0000x reference)
import functools

import numpy as np
import jax
import jax.numpy as jnp
from jax import lax
from jax.experimental import pallas as pl
from jax.experimental.pallas import tpu as pltpu
from jax.experimental.pallas import tpu_sc as plsc

D = 768
BATCH = 4
SEQ = 4096
NW = 32
POS_PER_W = SEQ // NW
PH = 2
POS_PER_PH = POS_PER_W // PH
PC = 8
C = PC * BATCH
CH_PER_PH = POS_PER_PH // PC
NCHUNK = PH * CH_PER_PH
NB = 3
LANES = 16
NJ = D // LANES
SCALE = float(np.sqrt(np.float32(D)))


def _sin_pe():
    position = np.arange(0, SEQ, dtype=np.float32)[:, None]
    div_term = np.exp(
        np.arange(0, D, 2).astype(np.float32) * (-np.log(10000.0) / D))
    pe = np.zeros((SEQ, D), dtype=np.float32)
    pe[:, 0::2] = np.sin(position * div_term)
    pe[:, 1::2] = np.cos(position * div_term)
    return pe


_PE_NP = _sin_pe()

_MESH = plsc.VectorSubcoreMesh(core_axis_name="c", subcore_axis_name="s")


@functools.partial(
    pl.kernel,
    mesh=_MESH,
    out_type=jax.ShapeDtypeStruct((BATCH * SEQ, D), jnp.float32),
    scratch_types=[
        pltpu.VMEM((NCHUNK, C), jnp.int32),
        pltpu.VMEM((POS_PER_PH, D), jnp.float32),
        pltpu.VMEM((C, D), jnp.float32),
        pltpu.VMEM((C, D), jnp.float32),
        pltpu.VMEM((C, D), jnp.float32),
        pltpu.SemaphoreType.DMA,
        pltpu.SemaphoreType.DMA,
    ],
)
def _embed(x_hbm, table_hbm, pe_hbm, out_hbm,
           idx_v, pe_v, rows0, rows1, rows2, gsem, ssem):
    cid = lax.axis_index("c")
    sid = lax.axis_index("s")
    wid = cid * 16 + sid
    pbase = wid * POS_PER_W
    bufs = (rows0, rows1, rows2)

    pltpu.sync_copy(x_hbm.at[wid], idx_v)

    def gather(k):
        return pltpu.async_copy(
            table_hbm.at[idx_v.at[k]], bufs[k % NB], gsem)

    gathers = [None] * NCHUNK
    gathers[0] = gather(0)
    gathers[1] = gather(1)

    pltpu.sync_copy(pe_hbm.at[pl.ds(pbase, POS_PER_PH)], pe_v)

    stores = [None] * NCHUNK
    for k in range(NCHUNK):
        ph, cpos = divmod(k, CH_PER_PH)
        if k == NCHUNK // 2:
            pltpu.sync_copy(
                pe_hbm.at[pl.ds(pbase + POS_PER_PH, POS_PER_PH)], pe_v)
        buf = bufs[k % NB]
        gathers[k].wait()

        @plsc.parallel_loop(0, NJ)
        def _(j, buf=buf, cpos=cpos):
            col = pl.ds(j * LANES, LANES)
            for p in range(PC):
                pe_vec = pe_v[cpos * PC + p, col]
                for b in range(BATCH):
                    r = b * PC + p
                    buf[r, col] = buf[r, col] * SCALE + pe_vec

        pos0 = pbase + ph * POS_PER_PH + cpos * PC
        stores[k] = [
            pltpu.async_copy(
                buf.at[pl.ds(b * PC, PC)],
                out_hbm.at[pl.ds(b * SEQ + pos0, PC)], ssem)
            for b in range(BATCH)
        ]
        if k >= 1:
            for cp in stores[k - 1]:
                cp.wait()
        if k + 2 < NCHUNK:
            gathers[k + 2] = gather(k + 2)

    for cp in stores[NCHUNK - 1]:
        cp.wait()


def kernel(x, table):
    xr = x.astype(jnp.int32).reshape(BATCH, NW, PH, CH_PER_PH, PC)
    xr = xr.transpose(1, 2, 3, 0, 4).reshape(NW, NCHUNK, C)
    out = _embed(xr, table, jnp.asarray(_PE_NP))
    return out.reshape(BATCH, SEQ, D)


# speedup vs baseline: 1.0520x; 1.0520x over previous
import functools

import numpy as np
import jax
import jax.numpy as jnp
from jax import lax
from jax.experimental import pallas as pl
from jax.experimental.pallas import tpu as pltpu
from jax.experimental.pallas import tpu_sc as plsc

D = 768
BATCH = 4
SEQ = 4096
NW = 32
POS_PER_W = SEQ // NW
PC = 16
NS = POS_PER_W // PC
LANES = 16
NJ = D // LANES
SCALE = float(np.sqrt(np.float32(D)))


def _sin_pe():
    position = np.arange(0, SEQ, dtype=np.float32)[:, None]
    div_term = np.exp(
        np.arange(0, D, 2).astype(np.float32) * (-np.log(10000.0) / D))
    pe = np.zeros((SEQ, D), dtype=np.float32)
    pe[:, 0::2] = np.sin(position * div_term)
    pe[:, 1::2] = np.cos(position * div_term)
    return pe


_PE_NP = _sin_pe()

_MESH = plsc.VectorSubcoreMesh(core_axis_name="c", subcore_axis_name="s")

_ROWBUF = [pltpu.VMEM((PC, D), jnp.float32) for _ in range(2 * BATCH)]
_PEBUF = [pltpu.VMEM((PC, D), jnp.float32) for _ in range(2)]


@functools.partial(
    pl.kernel,
    mesh=_MESH,
    out_type=jax.ShapeDtypeStruct((BATCH, SEQ, D), jnp.float32),
    scratch_types=[pltpu.VMEM((BATCH, POS_PER_W), jnp.int32)]
    + _ROWBUF + _PEBUF
    + [pltpu.SemaphoreType.DMA,
       pltpu.SemaphoreType.DMA,
       pltpu.SemaphoreType.DMA],
)
def _embed(x_hbm, table_hbm, pe_hbm, out_hbm, idx_v,
           r00, r01, r02, r03, r10, r11, r12, r13,
           pe0, pe1, gsem, ssem, psem):
    cid = lax.axis_index("c")
    sid = lax.axis_index("s")
    wid = cid * 16 + sid
    pbase = wid * POS_PER_W
    stages = ((r00, r01, r02, r03), (r10, r11, r12, r13))
    pebufs = (pe0, pe1)

    for b in range(BATCH):
        pltpu.sync_copy(x_hbm.at[b, wid], idx_v.at[b])

    def issue(s):
        bufs = stages[s % 2]
        g = [pltpu.async_copy(
                table_hbm.at[idx_v.at[b, pl.ds(s * PC, PC)]],
                bufs[b], gsem)
             for b in range(BATCH)]
        p = pltpu.async_copy(
            pe_hbm.at[pl.ds(pbase + s * PC, PC)], pebufs[s % 2], psem)
        return g, p

    gathers = [None] * NS
    stores = [None] * NS
    gathers[0] = issue(0)
    gathers[1] = issue(1)

    for s in range(NS):
        bufs = stages[s % 2]
        pe_v = pebufs[s % 2]
        g, p = gathers[s]
        for cp in g:
            cp.wait()
        p.wait()

        @plsc.parallel_loop(0, NJ)
        def _(j, bufs=bufs, pe_v=pe_v):
            col = pl.ds(j * LANES, LANES)
            for p_ in range(PC):
                pe_vec = pe_v[p_, col]
                for b in range(BATCH):
                    bufs[b][p_, col] = bufs[b][p_, col] * SCALE + pe_vec

        pos0 = pbase + s * PC
        stores[s] = [
            pltpu.async_copy(bufs[b], out_hbm.at[b, pl.ds(pos0, PC)], ssem)
            for b in range(BATCH)
        ]
        if s >= 1:
            for cp in stores[s - 1]:
                cp.wait()
        if s + 2 < NS:
            gathers[s + 2] = issue(s + 2)

    for cp in stores[NS - 1]:
        cp.wait()


def kernel(x, table):
    xr = x.astype(jnp.int32).reshape(BATCH, NW, POS_PER_W)
    return _embed(xr, table, jnp.asarray(_PE_NP))
